# baseline (speedup 1.0000x reference)
import functools

import jax
import jax.numpy as jnp
from jax import lax
from jax.experimental import pallas as pl
from jax.experimental.pallas import tpu as pltpu
from jax.experimental.pallas import tpu_sc as plsc

B = 16
L = 512
D = 384
T = 2048
NW = 32
B_SC = 2
FRAMES_PER_W = (B_SC * T) // NW
CHUNK = 64
NCH = FRAMES_PER_W // CHUNK
NBUF = 2
VL = 16


def _lr_body(xflat, dur, zrows, out, dur_v, counts_v, idx2d, buf0, buf1, zbuf,
             gsem0, gsem1, psem0, psem1):
    cid = lax.axis_index("c")
    sid = lax.axis_index("s")
    wid = sid * 2 + cid
    b = wid % B_SC
    seg = wid // B_SC

    dcp = pltpu.make_async_copy(dur.at[b], dur_v, gsem0)
    dcp.start()

    zeros16 = jnp.zeros((VL,), jnp.int32)
    for k in range(T // VL):
        counts_v[pl.ds(k * VL, VL)] = zeros16
    dcp.wait()

    ones16 = jnp.ones((VL,), jnp.int32)
    carry = jnp.int32(0)
    for k in range(L // VL):
        dch = dur_v[pl.ds(k * VL, VL)]
        r = jnp.maximum(dch, 1)
        cs = jnp.cumsum(r) + carry
        plsc.store_scatter(counts_v, [cs], ones16, mask=cs < T)
        carry = carry + jnp.sum(r)
    total = carry

    row0 = seg * NCH
    frame0 = seg * FRAMES_PER_W
    obase = b * T + frame0
    pl.when(total - (frame0 + (NCH - 1) * CHUNK) <= 0)(
        lambda: pltpu.sync_copy(zrows, zbuf))
    bufs = (buf0, buf1)
    gsems = (gsem0, gsem1)
    psems = (psem0, psem1)

    def gcopy(ci, p):
        return pltpu.make_async_copy(xflat.at[idx2d.at[row0 + ci]], bufs[p], gsems[p])

    def pvalid(ci, p):
        return pltpu.make_async_copy(
            bufs[p], out.at[pl.ds(obase + ci * CHUNK, CHUNK)], psems[p])

    def pzero(ci, p):
        return pltpu.make_async_copy(
            zbuf, out.at[pl.ds(obase + ci * CHUNK, CHUNK)], psems[p])

    def start_chunk(ci, p):
        v = total - (frame0 + ci * CHUNK)
        pl.when(v > 0)(lambda: gcopy(ci, p).start())

    def finish_chunk(ci, p):
        v = total - (frame0 + ci * CHUNK)

        def valid_case():
            gcopy(ci, p).wait()

            def zero_tail():
                def zero_row(rr, _):
                    for j in range(D // VL):
                        bufs[p][rr, pl.ds(j * VL, VL)] = jnp.zeros((VL,), jnp.float32)
                    return 0
                lax.fori_loop(v, CHUNK, zero_row, 0)
            pl.when(v < CHUNK)(zero_tail)
            pvalid(ci, p).start()

        def masked_case():
            pzero(ci, p).start()

        pl.when(v > 0)(valid_case)
        pl.when(v <= 0)(masked_case)

    base = b * L
    rpc = CHUNK // VL
    acc = jnp.int32(0)
    for k in range(T // VL):
        c = counts_v[pl.ds(k * VL, VL)]
        s = jnp.minimum(jnp.cumsum(c) + (acc + base), base + L - 1)
        idx2d[k // rpc, pl.ds((k % rpc) * VL, VL)] = s
        acc = acc + jnp.sum(c)

    for ci in range(NCH):
        p = ci % NBUF
        if ci >= NBUF:
            pvalid(ci - NBUF, p).wait()
        start_chunk(ci, p)
        if ci > 0:
            finish_chunk(ci - 1, (ci - 1) % NBUF)
    finish_chunk(NCH - 1, (NCH - 1) % NBUF)
    for ci in range(max(NCH - NBUF, 0), NCH):
        pvalid(ci, ci % NBUF).wait()


_lr_call = functools.partial(
    pl.kernel,
    out_type=jax.ShapeDtypeStruct((B_SC * T, D), jnp.float32),
    mesh=plsc.VectorSubcoreMesh(core_axis_name="c", subcore_axis_name="s"),
    compiler_params=pltpu.CompilerParams(needs_layout_passes=False),
    scratch_types=[
        pltpu.VMEM((L,), jnp.int32),
        pltpu.VMEM((T,), jnp.int32),
        pltpu.VMEM((T // CHUNK, CHUNK), jnp.int32),
        pltpu.VMEM((CHUNK, D), jnp.float32),
        pltpu.VMEM((CHUNK, D), jnp.float32),
        pltpu.VMEM((CHUNK, D), jnp.float32),
        pltpu.SemaphoreType.DMA,
        pltpu.SemaphoreType.DMA,
        pltpu.SemaphoreType.DMA,
        pltpu.SemaphoreType.DMA,
    ],
)(_lr_body)


TB = 512


def _tc_body(dur_ref, x_ref, out_ref):
    tbi = pl.program_id(1)
    d = dur_ref[0]
    r = jnp.maximum(d, 1).astype(jnp.float32)
    ii = lax.broadcasted_iota(jnp.int32, (L, L), 0)
    jj = lax.broadcasted_iota(jnp.int32, (L, L), 1)
    ut = (ii <= jj).astype(jnp.float32)
    cs = jnp.dot(r, ut, preferred_element_type=jnp.float32)
    csp = cs - r
    x = x_ref[0].astype(jnp.bfloat16)
    tt = (lax.broadcasted_iota(jnp.int32, (TB, 1), 0) + tbi * TB).astype(jnp.float32)
    oh = ((tt < cs).astype(jnp.bfloat16) - (tt < csp).astype(jnp.bfloat16))
    out_ref[0] = jnp.dot(oh, x, preferred_element_type=jnp.float32)


def _tc_call_full(x, dur):
    dur3 = dur.reshape(B, 1, L)
    return pl.pallas_call(
        _tc_body,
        grid=(B - B_SC, T // TB),
        in_specs=[
            pl.BlockSpec((1, 1, L), lambda b, t: (b + B_SC, 0, 0)),
            pl.BlockSpec((1, L, D), lambda b, t: (b + B_SC, 0, 0)),
        ],
        out_specs=pl.BlockSpec((1, TB, D), lambda b, t: (b + B_SC, t, 0)),
        out_shape=jax.ShapeDtypeStruct((B, T, D), jnp.float32),
    )(dur3, x)


def kernel(x, durations, target_len):
    dur = durations.astype(jnp.int32)
    xflat = x.reshape(B * L, D)
    zrows = jnp.zeros((CHUNK, D), jnp.float32)
    tc_full = _tc_call_full(x, dur)
    sc_out = _lr_call(xflat, dur, zrows)
    return tc_full.at[:B_SC].set(sc_out.reshape(B_SC, T, D))


# speedup vs baseline: 1.3957x; 1.3957x over previous
import functools

import jax
import jax.numpy as jnp
from jax import lax
from jax.experimental import pallas as pl
from jax.experimental.pallas import tpu as pltpu
from jax.experimental.pallas import tpu_sc as plsc

B = 16
L = 512
D = 384
T = 2048
NW = 32
B_SC = 2
FRAMES_PER_W = (B_SC * T) // NW
CHUNK = 64
NCH = FRAMES_PER_W // CHUNK
NBUF = 2
VL = 16


def _lr_body(xflat, dur, zrows, out, dur_v, counts_v, idx2d, buf0, buf1, zbuf,
             gsem0, gsem1, psem0, psem1):
    cid = lax.axis_index("c")
    sid = lax.axis_index("s")
    wid = sid * 2 + cid
    b = wid % B_SC
    seg = wid // B_SC

    dcp = pltpu.make_async_copy(dur.at[b], dur_v, gsem0)
    dcp.start()

    zeros16 = jnp.zeros((VL,), jnp.int32)
    for k in range(T // VL):
        counts_v[pl.ds(k * VL, VL)] = zeros16
    dcp.wait()

    ones16 = jnp.ones((VL,), jnp.int32)
    carry = jnp.int32(0)
    for k in range(L // VL):
        dch = dur_v[pl.ds(k * VL, VL)]
        r = jnp.maximum(dch, 1)
        cs = jnp.cumsum(r) + carry
        plsc.store_scatter(counts_v, [cs], ones16, mask=cs < T)
        carry = carry + jnp.sum(r)
    total = carry

    row0 = seg * NCH
    frame0 = seg * FRAMES_PER_W
    obase = b * T + frame0
    pl.when(total - (frame0 + (NCH - 1) * CHUNK) <= 0)(
        lambda: pltpu.sync_copy(zrows, zbuf))
    bufs = (buf0, buf1)
    gsems = (gsem0, gsem1)
    psems = (psem0, psem1)

    def gcopy(ci, p):
        return pltpu.make_async_copy(xflat.at[idx2d.at[row0 + ci]], bufs[p], gsems[p])

    def pvalid(ci, p):
        return pltpu.make_async_copy(
            bufs[p], out.at[pl.ds(obase + ci * CHUNK, CHUNK)], psems[p])

    def pzero(ci, p):
        return pltpu.make_async_copy(
            zbuf, out.at[pl.ds(obase + ci * CHUNK, CHUNK)], psems[p])

    def start_chunk(ci, p):
        v = total - (frame0 + ci * CHUNK)
        pl.when(v > 0)(lambda: gcopy(ci, p).start())

    def finish_chunk(ci, p):
        v = total - (frame0 + ci * CHUNK)

        def valid_case():
            gcopy(ci, p).wait()

            def zero_tail():
                def zero_row(rr, _):
                    for j in range(D // VL):
                        bufs[p][rr, pl.ds(j * VL, VL)] = jnp.zeros((VL,), jnp.float32)
                    return 0
                lax.fori_loop(v, CHUNK, zero_row, 0)
            pl.when(v < CHUNK)(zero_tail)
            pvalid(ci, p).start()

        def masked_case():
            pzero(ci, p).start()

        pl.when(v > 0)(valid_case)
        pl.when(v <= 0)(masked_case)

    base = b * L
    rpc = CHUNK // VL
    acc = jnp.int32(0)
    for k in range(T // VL):
        c = counts_v[pl.ds(k * VL, VL)]
        s = jnp.minimum(jnp.cumsum(c) + (acc + base), base + L - 1)
        idx2d[k // rpc, pl.ds((k % rpc) * VL, VL)] = s
        acc = acc + jnp.sum(c)

    for ci in range(NCH):
        p = ci % NBUF
        if ci >= NBUF:
            pvalid(ci - NBUF, p).wait()
        start_chunk(ci, p)
        if ci > 0:
            finish_chunk(ci - 1, (ci - 1) % NBUF)
    finish_chunk(NCH - 1, (NCH - 1) % NBUF)
    for ci in range(max(NCH - NBUF, 0), NCH):
        pvalid(ci, ci % NBUF).wait()


_lr_call = functools.partial(
    pl.kernel,
    out_type=jax.ShapeDtypeStruct((B_SC * T, D), jnp.float32),
    mesh=plsc.VectorSubcoreMesh(core_axis_name="c", subcore_axis_name="s"),
    compiler_params=pltpu.CompilerParams(needs_layout_passes=False),
    scratch_types=[
        pltpu.VMEM((L,), jnp.int32),
        pltpu.VMEM((T,), jnp.int32),
        pltpu.VMEM((T // CHUNK, CHUNK), jnp.int32),
        pltpu.VMEM((CHUNK, D), jnp.float32),
        pltpu.VMEM((CHUNK, D), jnp.float32),
        pltpu.VMEM((CHUNK, D), jnp.float32),
        pltpu.SemaphoreType.DMA,
        pltpu.SemaphoreType.DMA,
        pltpu.SemaphoreType.DMA,
        pltpu.SemaphoreType.DMA,
    ],
)(_lr_body)


TB = 512


def _tc_body(dur_ref, x_ref, out_ref):
    d = dur_ref[0]
    r = jnp.maximum(d, 1).astype(jnp.float32)
    ii = lax.broadcasted_iota(jnp.int32, (L, L), 0)
    jj = lax.broadcasted_iota(jnp.int32, (L, L), 1)
    ut = (ii <= jj).astype(jnp.float32)
    cs = jnp.dot(r, ut, preferred_element_type=jnp.float32)
    csp = cs - r
    x = x_ref[0].astype(jnp.bfloat16)
    for tb in range(T // TB):
        tt = (lax.broadcasted_iota(jnp.int32, (TB, 1), 0) + tb * TB).astype(jnp.float32)
        oh = ((tt < cs).astype(jnp.bfloat16) - (tt < csp).astype(jnp.bfloat16))
        out_ref[0, tb * TB:(tb + 1) * TB, :] = jnp.dot(
            oh, x, preferred_element_type=jnp.float32)


def _tc_call_full(x, dur):
    dur3 = dur.reshape(B, 1, L)
    return pl.pallas_call(
        _tc_body,
        grid=(B - B_SC,),
        in_specs=[
            pl.BlockSpec((1, 1, L), lambda b: (b + B_SC, 0, 0)),
            pl.BlockSpec((1, L, D), lambda b: (b + B_SC, 0, 0)),
        ],
        out_specs=pl.BlockSpec((1, T, D), lambda b: (b + B_SC, 0, 0)),
        out_shape=jax.ShapeDtypeStruct((B, T, D), jnp.float32),
    )(dur3, x)


def kernel(x, durations, target_len):
    dur = durations.astype(jnp.int32)
    xflat = x.reshape(B * L, D)
    zrows = jnp.zeros((CHUNK, D), jnp.float32)
    tc_full = _tc_call_full(x, dur)
    sc_out = _lr_call(xflat, dur, zrows)
    return tc_full.at[:B_SC].set(sc_out.reshape(B_SC, T, D))
